# baseline (speedup 1.0000x reference)
import functools

import jax
import jax.numpy as jnp
from jax import lax
from jax.experimental import pallas as pl
from jax.experimental.pallas import tpu as pltpu
from jax.experimental.pallas import tpu_sc as plsc

NUM_USERS = 200000
LANES = 16
CHUNK = 128


@functools.cache
def _build_pair_gather(B, P, NC, NS):
    NW = NC * NS
    b_per_w = B // NW
    n_chunks = b_per_w // CHUNK
    n_groups = b_per_w // LANES
    mesh = plsc.VectorSubcoreMesh(core_axis_name="c", subcore_axis_name="s")

    @functools.partial(
        pl.kernel,
        mesh=mesh,
        compiler_params=pltpu.CompilerParams(
            needs_layout_passes=False, use_tc_tiling_on_sc=True),
        out_type=jax.ShapeDtypeStruct((B, P), jnp.float32),
        scratch_types=[
            pltpu.VMEM((b_per_w,), jnp.int32),
            pltpu.VMEM((b_per_w,), jnp.int32),
            pltpu.VMEM((b_per_w, P), jnp.float32),
            pltpu.SemaphoreType.DMA,
        ],
    )
    def k1(e_hbm, name2_hbm, out_hbm, e_v, p_v, pr_v, sem):
        wid = lax.axis_index("s") * NC + lax.axis_index("c")
        base = wid * b_per_w
        pltpu.sync_copy(e_hbm.at[pl.ds(base, b_per_w)], e_v)

        def pidx(j, carry):
            sl = pl.ds(j * LANES, LANES)
            p_v[sl] = lax.shift_right_logical(e_v[sl], 1)
            return carry

        lax.fori_loop(0, n_groups, pidx, 0)

        descs = []
        for k in range(n_chunks):
            sl = pl.ds(k * CHUNK, CHUNK)
            descs.append(pltpu.async_copy(
                name2_hbm.at[p_v.at[sl]], pr_v.at[sl], sem))
        for dsc in descs:
            dsc.wait()
        pltpu.sync_copy(pr_v, out_hbm.at[pl.ds(base, b_per_w)])

    return k1


@functools.cache
def _build_combine(B, D, NC, NS):
    NW = NC * NS
    P = 2 * D
    b_per_w = B // NW
    n_chunks = b_per_w // CHUNK
    n_groups = b_per_w // LANES
    n_dchunks = D // LANES
    zero_row = 68
    mesh = plsc.VectorSubcoreMesh(core_axis_name="c", subcore_axis_name="s")

    @functools.partial(
        pl.kernel,
        mesh=mesh,
        compiler_params=pltpu.CompilerParams(
            needs_layout_passes=False, use_tc_tiling_on_sc=False),
        out_type=jax.ShapeDtypeStruct((B, D), jnp.float32),
        scratch_types=[
            pltpu.VMEM((b_per_w,), jnp.int32),
            pltpu.VMEM((4, b_per_w), jnp.int32),
            pltpu.VMEM((b_per_w, P), jnp.float32),
            pltpu.VMEM((b_per_w, D), jnp.float32),
            pltpu.VMEM((69, D), jnp.float32),
            pltpu.SemaphoreType.DMA,
        ],
    )
    def k2(e_hbm, x3_hbm, x4_hbm, x5_hbm, x6_hbm, ft_hbm, pairs_hbm,
           out_hbm, e_v, xc_v, pr_v, acc_v, ft_v, sem):
        wid = lax.axis_index("s") * NC + lax.axis_index("c")
        base = wid * b_per_w

        pltpu.sync_copy(e_hbm.at[wid], e_v)
        pltpu.sync_copy(ft_hbm, ft_v)
        pltpu.sync_copy(pairs_hbm.at[pl.ds(base, b_per_w)], pr_v)

        descs = []
        for k in range(n_chunks):
            idx = e_v.at[pl.ds(k * CHUNK, CHUNK)]
            for c, xh in enumerate((x3_hbm, x4_hbm, x5_hbm, x6_hbm)):
                descs.append(pltpu.async_copy(
                    xh.at[idx], xc_v.at[c, pl.ds(k * CHUNK, CHUNK)], sem))
        for dsc in descs:
            dsc.wait()

        def group(g, carry):
            gbase = g * LANES
            ev = e_v[pl.ds(gbase, LANES)]
            mask = ev >= NUM_USERS
            half = (ev & 1) * D
            f3 = jnp.where(mask, xc_v[0, pl.ds(gbase, LANES)] + 46, zero_row)
            f4 = jnp.where(mask, xc_v[1, pl.ds(gbase, LANES)] + 35, zero_row)
            f5 = jnp.where(mask, xc_v[2, pl.ds(gbase, LANES)], zero_row)
            f6 = jnp.where(mask, xc_v[3, pl.ds(gbase, LANES)] + 9, zero_row)
            for l in range(LANES):
                r = gbase + l
                s3, s4, s5, s6 = f3[l], f4[l], f5[l], f6[l]
                off = half[l]
                for c in range(n_dchunks):
                    dcol = pl.ds(c * LANES, LANES)
                    acc_v[r, dcol] = (pr_v[r, pl.ds(off + c * LANES, LANES)]
                                      + ft_v[s5, dcol] + ft_v[s6, dcol]
                                      + ft_v[s4, dcol] + ft_v[s3, dcol])
            return carry

        lax.fori_loop(0, n_groups, group, 0)
        pltpu.sync_copy(acc_v, out_hbm.at[pl.ds(base, b_per_w)])

    return k2


def kernel(e, x, feat_table, name_emb):
    B = e.shape[0]
    D = feat_table.shape[1]
    info = plsc.get_sparse_core_info()
    NC, NS = info.num_cores, info.num_subcores
    NW = NC * NS
    e1 = e.astype(jnp.int32)
    e2 = e1.reshape(NW, B // NW)
    xi = x.astype(jnp.int32)
    x3, x4, x5, x6 = xi[:, 3], xi[:, 4], xi[:, 5], xi[:, 6]
    ftp = jnp.concatenate(
        [feat_table, jnp.zeros((1, D), feat_table.dtype)], axis=0)
    name2 = name_emb.reshape(name_emb.shape[0] // 2, 2 * D)
    pairs = _build_pair_gather(B, 2 * D, NC, NS)(e1, name2)
    return _build_combine(B, D, NC, NS)(e2, x3, x4, x5, x6, ftp, pairs)
